# baseline (speedup 1.0000x reference)
import jax
import jax.numpy as jnp
from jax.experimental import pallas as pl
from jax.experimental.pallas import tpu as pltpu
from jax.experimental.pallas import tpu_sc as plsc

_WINDOW = 128


def kernel(token_ids, matrix):
    b, s = token_ids.shape
    n, d = matrix.shape
    num_indices = b * s
    indices = token_ids.astype(jnp.int32).reshape(1, num_indices)

    mesh = plsc.VectorSubcoreMesh(
        core_axis_name="core", subcore_axis_name="subcore"
    )

    @pl.kernel(
        out_type=jax.ShapeDtypeStruct((num_indices, d), matrix.dtype),
        mesh=mesh,
    )
    def gather_kernel(x_hbm, i_hbm, o_hbm):
        def body(i_vmem, o_vmem):
            pltpu.sync_copy(x_hbm.at[i_vmem.at[0]], o_vmem)

        pltpu.emit_pipeline(
            body,
            grid=(num_indices // _WINDOW,),
            in_specs=[pl.BlockSpec((1, _WINDOW), index_map=lambda i: (0, i))],
            out_specs=[pl.BlockSpec((_WINDOW, d), index_map=lambda i: (i, 0))],
            core_axis_name=("core", "subcore"),
            dimension_semantics=(pltpu.PARALLEL,),
        )(i_hbm, o_hbm)

    return gather_kernel(matrix, indices).reshape(b, s, d)


# speedup vs baseline: 1.0625x; 1.0625x over previous
import jax
import jax.numpy as jnp
from jax.experimental import pallas as pl
from jax.experimental.pallas import tpu as pltpu
from jax.experimental.pallas import tpu_sc as plsc

_WINDOW = 256


def kernel(token_ids, matrix):
    b, s = token_ids.shape
    n, d = matrix.shape
    num_indices = b * s
    indices = token_ids.astype(jnp.int32).reshape(1, num_indices)

    mesh = plsc.VectorSubcoreMesh(
        core_axis_name="core", subcore_axis_name="subcore"
    )

    @pl.kernel(
        out_type=jax.ShapeDtypeStruct((num_indices, d), matrix.dtype),
        mesh=mesh,
    )
    def gather_kernel(x_hbm, i_hbm, o_hbm):
        def body(i_vmem, o_vmem):
            pltpu.sync_copy(x_hbm.at[i_vmem.at[0]], o_vmem)

        pltpu.emit_pipeline(
            body,
            grid=(num_indices // _WINDOW,),
            in_specs=[pl.BlockSpec((1, _WINDOW), index_map=lambda i: (0, i))],
            out_specs=[pl.BlockSpec((_WINDOW, d), index_map=lambda i: (i, 0))],
            core_axis_name=("core", "subcore"),
            dimension_semantics=(pltpu.PARALLEL,),
        )(i_hbm, o_hbm)

    return gather_kernel(matrix, indices).reshape(b, s, d)


# speedup vs baseline: 1.3590x; 1.2791x over previous
import jax
import jax.numpy as jnp
from jax.experimental import pallas as pl
from jax.experimental.pallas import tpu as pltpu
from jax.experimental.pallas import tpu_sc as plsc

_BBLK = 8


def kernel(token_ids, matrix):
    b, s = token_ids.shape
    n, d = matrix.shape
    nblocks = b // _BBLK
    indices = token_ids.astype(jnp.int32).reshape(nblocks, _BBLK, s)

    mesh = plsc.VectorSubcoreMesh(
        core_axis_name="core", subcore_axis_name="subcore"
    )

    @pl.kernel(
        out_type=jax.ShapeDtypeStruct((b, s, d), matrix.dtype),
        mesh=mesh,
    )
    def gather_kernel(x_hbm, i_hbm, o_hbm):
        def body(i_vmem, o_vmem):
            @pl.loop(0, _BBLK)
            def _(j):
                pltpu.sync_copy(
                    x_hbm.at[i_vmem.at[0, j]],
                    o_vmem.at[j],
                )

        pltpu.emit_pipeline(
            body,
            grid=(nblocks,),
            in_specs=[
                pl.BlockSpec((1, _BBLK, s), index_map=lambda i: (i, 0, 0))
            ],
            out_specs=[
                pl.BlockSpec((_BBLK, s, d), index_map=lambda i: (i, 0, 0))
            ],
            core_axis_name=("core", "subcore"),
            dimension_semantics=(pltpu.PARALLEL,),
        )(i_hbm, o_hbm)

    return gather_kernel(matrix, indices)


# speedup vs baseline: 1.8645x; 1.3719x over previous
import jax
import jax.numpy as jnp
from jax.experimental import pallas as pl
from jax.experimental.pallas import tpu as pltpu
from jax.experimental.pallas import tpu_sc as plsc

_BBLK = 8


def kernel(token_ids, matrix):
    b, s = token_ids.shape
    n, d = matrix.shape
    nblocks = b // _BBLK
    indices = token_ids.astype(jnp.int32).reshape(nblocks, 1, _BBLK * s)

    mesh = plsc.VectorSubcoreMesh(
        core_axis_name="core", subcore_axis_name="subcore"
    )

    @pl.kernel(
        out_type=jax.ShapeDtypeStruct((b, s, d), matrix.dtype),
        mesh=mesh,
    )
    def gather_kernel(x_hbm, i_hbm, o_hbm):
        def body(i_vmem, o_vmem):
            pltpu.sync_copy(
                x_hbm.at[i_vmem.at[0, 0]],
                o_vmem.reshape(_BBLK * s, d),
            )

        pltpu.emit_pipeline(
            body,
            grid=(nblocks,),
            in_specs=[
                pl.BlockSpec((1, 1, _BBLK * s), index_map=lambda i: (i, 0, 0))
            ],
            out_specs=[
                pl.BlockSpec((_BBLK, s, d), index_map=lambda i: (i, 0, 0))
            ],
            core_axis_name=("core", "subcore"),
            dimension_semantics=(pltpu.PARALLEL,),
        )(i_hbm, o_hbm)

    return gather_kernel(matrix, indices)


# speedup vs baseline: 1.8655x; 1.0005x over previous
import jax
import jax.numpy as jnp
from jax.experimental import pallas as pl
from jax.experimental.pallas import tpu as pltpu
from jax.experimental.pallas import tpu_sc as plsc

_BBLK = 8


def kernel(token_ids, matrix):
    b, s = token_ids.shape
    n, d = matrix.shape
    nblocks = b // _BBLK
    indices = token_ids.astype(jnp.int32).reshape(nblocks, 1, _BBLK * s)

    mesh = plsc.VectorSubcoreMesh(
        core_axis_name="core", subcore_axis_name="subcore"
    )

    @pl.kernel(
        out_type=jax.ShapeDtypeStruct((b, s, d), matrix.dtype),
        mesh=mesh,
    )
    def gather_kernel(x_hbm, i_hbm, o_hbm):
        def body(i_vmem, o_vmem):
            pltpu.sync_copy(
                x_hbm.at[i_vmem.at[0, 0]],
                o_vmem.reshape(_BBLK * s, d),
            )

        pltpu.emit_pipeline(
            body,
            grid=(nblocks,),
            in_specs=[
                pl.BlockSpec((1, 1, _BBLK * s), index_map=lambda i: (i, 0, 0))
            ],
            out_specs=[
                pl.BlockSpec((_BBLK, s, d), index_map=lambda i: (i, 0, 0))
            ],
            core_axis_name=("core", "subcore"),
            dimension_semantics=(pltpu.PARALLEL,),
            trace_scopes=False,
        )(i_hbm, o_hbm)

    return gather_kernel(matrix, indices)
